# Initial kernel scaffold; baseline (speedup 1.0000x reference)
import jax
import jax.numpy as jnp
from jax.experimental import pallas as pl


def kernel(x, edge_index, branch_child, time_value, is_root, W1, b1, W2, b2, Wh1, bh1, Wh2, bh2, Wh3, bh3):
    raise NotImplementedError("write your pallas kernel here")


# speedup vs baseline: 2.8990x; 2.8990x over previous
import functools

import jax
import jax.numpy as jnp
from jax import lax
from jax.experimental import pallas as pl

N_NODES = 10000
N_PAD = 10016
N_EDGES = 320000
HID = 64
IN_DIM = 128
N_ACT = 4096
FOCAL = 5

_HIGH = jax.lax.Precision.HIGHEST
_INTERPRET = False


def _rowmask(nrows):
    r = lax.broadcasted_iota(jnp.int32, (nrows, 1), 0)
    return (r <= N_NODES).astype(jnp.float32)


def _tc1_body(x_ref, w1_ref, dpart_ref, hws_ref, dinv_ref):
    deg = dpart_ref[0, :, 0:1] + dpart_ref[1, :, 0:1] + 1.0
    dinv = lax.rsqrt(jnp.clip(deg, 1.0, None))
    dinv_ref[...] = dinv
    hw = jnp.dot(x_ref[...], w1_ref[...], preferred_element_type=jnp.float32,
                 precision=_HIGH)
    hws_ref[0:N_NODES, :] = dinv[0:N_NODES] * hw
    foc = w1_ref[FOCAL:FOCAL + 1, :] + w1_ref[IN_DIM - 1:IN_DIM, :]
    tail = lax.broadcasted_iota(jnp.int32, (N_PAD - N_NODES, 1), 0)
    focmask = (tail == 0).astype(jnp.float32)
    hws_ref[N_NODES:N_PAD, :] = focmask * (dinv[N_NODES:N_NODES + 1] * foc)


def _tc1(x, w1, dpart):
    return pl.pallas_call(
        _tc1_body,
        out_shape=(jax.ShapeDtypeStruct((N_PAD, HID), jnp.float32),
                   jax.ShapeDtypeStruct((N_PAD, 1), jnp.float32)),
        interpret=_INTERPRET,
    )(x, w1, dpart)


def _tc2_body(agg_ref, hws1_ref, dinv_ref, b1_ref, w2_ref, hws2_ref):
    pre = agg_ref[0] + agg_ref[1] + hws1_ref[...]
    h = jnp.maximum(dinv_ref[...] * pre + b1_ref[...], 0.0) * _rowmask(N_PAD)
    hw2 = jnp.dot(h, w2_ref[...], preferred_element_type=jnp.float32,
                  precision=_HIGH)
    hws2_ref[...] = dinv_ref[...] * hw2


def _tc2(agg, hws1, dinv, b1, w2):
    return pl.pallas_call(
        _tc2_body,
        out_shape=jax.ShapeDtypeStruct((N_PAD, HID), jnp.float32),
        interpret=_INTERPRET,
    )(agg, hws1, dinv, b1.reshape(1, HID), w2)


def _tc3_body(agg_ref, hws2_ref, dinv_ref, b2_ref, emb_ref):
    pre = agg_ref[0] + agg_ref[1] + hws2_ref[...]
    emb_ref[...] = (dinv_ref[...] * pre + b2_ref[...]) * _rowmask(N_PAD)


def _tc3(agg, hws2, dinv, b2):
    return pl.pallas_call(
        _tc3_body,
        out_shape=jax.ShapeDtypeStruct((N_PAD, HID), jnp.float32),
        interpret=_INTERPRET,
    )(agg, hws2, dinv, b2.reshape(1, HID))


def _tc4_body(emb_ref, ht_ref, tn_ref, ir_ref, wh1_ref, bh1_ref, wh2_ref,
              bh2_ref, wh3_ref, bh3_ref, ef_ref, log_ref, prob_ref):
    hf = emb_ref[N_NODES:N_NODES + 1, :]
    ht = ht_ref[...]
    hfb = jnp.broadcast_to(hf, ht.shape)
    dabs = jnp.abs(hfb - ht)
    prod = hfb * ht
    tn = tn_ref[...] * (1.0 / (1.0 + 1e-08))
    ir = ir_ref[...]
    ef_ref[...] = jnp.concatenate([hfb, ht, dabs, prod, tn, ir], axis=1)

    A = wh1_ref[0:HID, :]
    B = wh1_ref[HID:2 * HID, :]
    C = wh1_ref[2 * HID:3 * HID, :]
    D = wh1_ref[3 * HID:4 * HID, :]
    wt = wh1_ref[4 * HID:4 * HID + 1, :]
    wr = wh1_ref[4 * HID + 1:4 * HID + 2, :]
    bias1 = bh1_ref[...] + jnp.dot(hf, A, preferred_element_type=jnp.float32,
                                   precision=_HIGH)
    z = (jnp.dot(ht, B, preferred_element_type=jnp.float32, precision=_HIGH)
         + jnp.dot(dabs, C, preferred_element_type=jnp.float32,
                   precision=_HIGH)
         + jnp.dot(prod, D, preferred_element_type=jnp.float32,
                   precision=_HIGH)
         + tn * wt + ir * wr + bias1)
    z = jnp.where(z > 0, z, jnp.exp(z) - 1.0)
    z = jnp.dot(z, wh2_ref[...], preferred_element_type=jnp.float32,
                precision=_HIGH) + bh2_ref[...]
    z = jnp.where(z > 0, z, jnp.exp(z) - 1.0)
    l = jnp.dot(z, wh3_ref[...], preferred_element_type=jnp.float32,
                precision=_HIGH) + bh3_ref[0, 0]
    log_ref[...] = l
    m = jnp.max(l)
    e = jnp.exp(l - m)
    prob_ref[...] = e / jnp.sum(e)


def _tc4(emb_pad, ht, tv, ir, wh1, bh1, wh2, bh2, wh3, bh3):
    return pl.pallas_call(
        _tc4_body,
        out_shape=(jax.ShapeDtypeStruct((N_ACT, 4 * HID + 2), jnp.float32),
                   jax.ShapeDtypeStruct((N_ACT, 1), jnp.float32),
                   jax.ShapeDtypeStruct((N_ACT, 1), jnp.float32)),
        interpret=_INTERPRET,
    )(emb_pad, ht, tv.reshape(N_ACT, 1), ir.reshape(N_ACT, 1),
      wh1, bh1.reshape(1, HID), wh2, bh2.reshape(1, HID), wh3,
      bh3.reshape(1, 1))


def kernel(x, edge_index, branch_child, time_value, is_root, W1, b1, W2, b2,
           Wh1, bh1, Wh2, bh2, Wh3, bh3):
    s_all = jnp.concatenate([edge_index[0], edge_index[1]])
    d_all = jnp.concatenate([edge_index[1], edge_index[0]])

    degc = jnp.zeros((N_PAD,), jnp.float32).at[d_all].add(1.0)
    dpart = jnp.zeros((2, N_PAD, 16), jnp.float32).at[0, :, 0].set(degc)

    hws1, dinv = _tc1(x, W1, dpart)

    agg = jnp.zeros((N_PAD, HID), jnp.float32).at[d_all].add(hws1[s_all])
    agg1 = jnp.stack([agg, jnp.zeros_like(agg)])

    hws2 = _tc2(agg1, hws1, dinv, b1, W2)

    agg_2 = jnp.zeros((N_PAD, HID), jnp.float32).at[d_all].add(hws2[s_all])
    agg2 = jnp.stack([agg_2, jnp.zeros_like(agg_2)])

    emb_pad = _tc3(agg2, hws2, dinv, b2)

    ht = emb_pad[branch_child]

    ef, logits, probs = _tc4(emb_pad, ht, time_value, is_root,
                             Wh1, bh1, Wh2, bh2, Wh3, bh3)

    emb = emb_pad[:N_NODES + 1]
    leaf_feature = jnp.zeros((126,), jnp.float32).at[FOCAL].set(1.0)
    return (logits[:, 0], probs[:, 0], ef, emb, leaf_feature)


# speedup vs baseline: 23.7314x; 8.1862x over previous
import functools

import jax
import jax.numpy as jnp
from jax import lax
from jax.experimental import pallas as pl
from jax.experimental.pallas import tpu as pltpu
from jax.experimental.pallas import tpu_sc as plsc

N_NODES = 10000
N_PAD = 10112
PAD_IDX = N_NODES + 1
N_EDGES = 320000
HID = 64
IN_DIM = 128
N_ACT = 4096
FOCAL = 5

NC = 2
NT = 16
NW = NC * NT
ROWS_PER_TILE = N_PAD // NT
CHUNK = 128
K_REAL = 158
K_ALLOC = 160
E_SLOTS = NW * K_REAL * CHUNK

_HIGH = jax.lax.Precision.HIGHEST
_INTERPRET = False

_SC_MESH = plsc.VectorSubcoreMesh(core_axis_name="c", subcore_axis_name="s")


def _rowmask(nrows):
    r = lax.broadcasted_iota(jnp.int32, (nrows, 1), 0)
    return (r <= N_NODES).astype(jnp.float32)


def _sc_deg_body(d_hbm, ones_hbm, z_hbm, out_hbm, d_v, dbuf, ones_v, sem,
                 acc):
    cid = lax.axis_index("c")
    sid = lax.axis_index("s")
    wid = cid * NT + sid
    row0 = sid * ROWS_PER_TILE
    pltpu.sync_copy(d_hbm.at[wid], d_v)
    pltpu.sync_copy(ones_hbm, ones_v)
    pltpu.sync_copy(z_hbm.at[pl.ds(row0, ROWS_PER_TILE)],
                    acc.at[pl.ds(row0, ROWS_PER_TILE)])
    plsc.subcore_barrier()

    def step(j, carry):
        for i in range(CHUNK // 16):
            dbuf[pl.ds(16 * i, 16)] = d_v[j, pl.ds(16 * i, 16)]
        pltpu.sync_copy(ones_v, acc.at[dbuf], add=True)
        return carry

    lax.fori_loop(0, K_REAL, step, 0)
    plsc.subcore_barrier()
    pltpu.sync_copy(acc.at[pl.ds(row0, ROWS_PER_TILE)],
                    out_hbm.at[cid, pl.ds(row0, ROWS_PER_TILE)])


def _sc_deg(d_arr, ones16, z16):
    return pl.kernel(
        _sc_deg_body,
        out_type=jax.ShapeDtypeStruct((NC, N_PAD, 16), jnp.float32),
        mesh=_SC_MESH,
        compiler_params=pltpu.CompilerParams(use_tc_tiling_on_sc=False),
        scratch_types=[
            pltpu.VMEM((K_ALLOC, CHUNK), jnp.int32),
            pltpu.VMEM((CHUNK,), jnp.int32),
            pltpu.VMEM((CHUNK, 16), jnp.float32),
            pltpu.SemaphoreType.DMA,
            pltpu.VMEM_SHARED((N_PAD, 16), jnp.float32),
        ],
    )(d_arr, ones16, z16)


def _sc_spmm_body(s_hbm, d_hbm, tbl_hbm, z_hbm, out_hbm, s_v, d_v, dbuf,
                  buf_a, buf_b, sem_a, sem_b, acc):
    cid = lax.axis_index("c")
    sid = lax.axis_index("s")
    wid = cid * NT + sid
    row0 = sid * ROWS_PER_TILE
    pltpu.sync_copy(s_hbm.at[wid], s_v)
    pltpu.sync_copy(d_hbm.at[wid], d_v)
    pltpu.sync_copy(z_hbm.at[pl.ds(row0, ROWS_PER_TILE)],
                    acc.at[pl.ds(row0, ROWS_PER_TILE)])
    plsc.subcore_barrier()

    pltpu.async_copy(tbl_hbm.at[s_v.at[0]], buf_a, sem_a)

    def pair(g, carry):
        j0 = 2 * g
        j1 = j0 + 1
        pltpu.async_copy(tbl_hbm.at[s_v.at[j1]], buf_b, sem_b)
        pltpu.make_async_copy(tbl_hbm.at[s_v.at[j0]], buf_a, sem_a).wait()
        for i in range(CHUNK // 16):
            dbuf[pl.ds(16 * i, 16)] = d_v[j0, pl.ds(16 * i, 16)]
        pltpu.sync_copy(buf_a, acc.at[dbuf], add=True)
        pltpu.async_copy(tbl_hbm.at[s_v.at[j0 + 2]], buf_a, sem_a)
        pltpu.make_async_copy(tbl_hbm.at[s_v.at[j1]], buf_b, sem_b).wait()
        for i in range(CHUNK // 16):
            dbuf[pl.ds(16 * i, 16)] = d_v[j1, pl.ds(16 * i, 16)]
        pltpu.sync_copy(buf_b, acc.at[dbuf], add=True)
        return carry

    lax.fori_loop(0, K_REAL // 2, pair, 0)
    pltpu.make_async_copy(tbl_hbm.at[s_v.at[K_REAL]], buf_a, sem_a).wait()
    plsc.subcore_barrier()
    pltpu.sync_copy(acc.at[pl.ds(row0, ROWS_PER_TILE)],
                    out_hbm.at[cid, pl.ds(row0, ROWS_PER_TILE)])


def _sc_spmm(s_arr, d_arr, tbl, z64):
    return pl.kernel(
        _sc_spmm_body,
        out_type=jax.ShapeDtypeStruct((NC, N_PAD, HID), jnp.float32),
        mesh=_SC_MESH,
        compiler_params=pltpu.CompilerParams(use_tc_tiling_on_sc=False),
        scratch_types=[
            pltpu.VMEM((K_ALLOC, CHUNK), jnp.int32),
            pltpu.VMEM((K_ALLOC, CHUNK), jnp.int32),
            pltpu.VMEM((CHUNK,), jnp.int32),
            pltpu.VMEM((CHUNK, HID), jnp.float32),
            pltpu.VMEM((CHUNK, HID), jnp.float32),
            pltpu.SemaphoreType.DMA,
            pltpu.SemaphoreType.DMA,
            pltpu.VMEM_SHARED((N_PAD, HID), jnp.float32),
        ],
    )(s_arr, d_arr, tbl, z64)


def _sc_gather_body(bc_hbm, tbl_hbm, out_hbm, idx_v, buf, sem):
    cid = lax.axis_index("c")
    sid = lax.axis_index("s")
    wid = cid * NT + sid
    pltpu.sync_copy(bc_hbm.at[pl.ds(wid * CHUNK, CHUNK)], idx_v)
    pltpu.async_copy(tbl_hbm.at[idx_v], buf, sem).wait()
    pltpu.sync_copy(buf, out_hbm.at[pl.ds(wid * CHUNK, CHUNK)])


def _sc_gather(bc, tbl):
    return pl.kernel(
        _sc_gather_body,
        out_type=jax.ShapeDtypeStruct((N_ACT, HID), jnp.float32),
        mesh=_SC_MESH,
        compiler_params=pltpu.CompilerParams(use_tc_tiling_on_sc=False),
        scratch_types=[
            pltpu.VMEM((CHUNK,), jnp.int32),
            pltpu.VMEM((CHUNK, HID), jnp.float32),
            pltpu.SemaphoreType.DMA,
        ],
    )(bc, tbl)


def _tc1_body(x_ref, w1_ref, dpart_ref, hws_ref, dinv_ref):
    deg = dpart_ref[0, :, 0:1] + dpart_ref[1, :, 0:1] + 1.0
    dinv = lax.rsqrt(jnp.clip(deg, 1.0, None))
    dinv_ref[...] = dinv
    hw = jnp.dot(x_ref[...], w1_ref[...], preferred_element_type=jnp.float32,
                 precision=_HIGH)
    hws_ref[0:N_NODES, :] = dinv[0:N_NODES] * hw
    foc = w1_ref[FOCAL:FOCAL + 1, :] + w1_ref[IN_DIM - 1:IN_DIM, :]
    tail = lax.broadcasted_iota(jnp.int32, (N_PAD - N_NODES, 1), 0)
    focmask = (tail == 0).astype(jnp.float32)
    hws_ref[N_NODES:N_PAD, :] = focmask * (dinv[N_NODES:N_NODES + 1] * foc)


def _tc1(x, w1, dpart):
    return pl.pallas_call(
        _tc1_body,
        out_shape=(jax.ShapeDtypeStruct((N_PAD, HID), jnp.float32),
                   jax.ShapeDtypeStruct((N_PAD, 1), jnp.float32)),
        interpret=_INTERPRET,
    )(x, w1, dpart)


def _tc2_body(agg_ref, hws1_ref, dinv_ref, b1_ref, w2_ref, hws2_ref):
    pre = agg_ref[0] + agg_ref[1] + hws1_ref[...]
    h = jnp.maximum(dinv_ref[...] * pre + b1_ref[...], 0.0) * _rowmask(N_PAD)
    hw2 = jnp.dot(h, w2_ref[...], preferred_element_type=jnp.float32,
                  precision=_HIGH)
    hws2_ref[...] = dinv_ref[...] * hw2


def _tc2(agg, hws1, dinv, b1, w2):
    return pl.pallas_call(
        _tc2_body,
        out_shape=jax.ShapeDtypeStruct((N_PAD, HID), jnp.float32),
        interpret=_INTERPRET,
    )(agg, hws1, dinv, b1.reshape(1, HID), w2)


def _tc3_body(agg_ref, hws2_ref, dinv_ref, b2_ref, emb_ref):
    pre = agg_ref[0] + agg_ref[1] + hws2_ref[...]
    emb_ref[...] = (dinv_ref[...] * pre + b2_ref[...]) * _rowmask(N_PAD)


def _tc3(agg, hws2, dinv, b2):
    return pl.pallas_call(
        _tc3_body,
        out_shape=jax.ShapeDtypeStruct((N_PAD, HID), jnp.float32),
        interpret=_INTERPRET,
    )(agg, hws2, dinv, b2.reshape(1, HID))


def _tc4_body(emb_ref, ht_ref, tn_ref, ir_ref, wh1_ref, bh1_ref, wh2_ref,
              bh2_ref, wh3_ref, bh3_ref, ef_ref, log_ref, prob_ref):
    hf = emb_ref[N_NODES:N_NODES + 1, :]
    ht = ht_ref[...]
    hfb = jnp.broadcast_to(hf, ht.shape)
    dabs = jnp.abs(hfb - ht)
    prod = hfb * ht
    tn = tn_ref[...] * (1.0 / (1.0 + 1e-08))
    ir = ir_ref[...]
    ef_ref[...] = jnp.concatenate([hfb, ht, dabs, prod, tn, ir], axis=1)

    A = wh1_ref[0:HID, :]
    B = wh1_ref[HID:2 * HID, :]
    C = wh1_ref[2 * HID:3 * HID, :]
    D = wh1_ref[3 * HID:4 * HID, :]
    wt = wh1_ref[4 * HID:4 * HID + 1, :]
    wr = wh1_ref[4 * HID + 1:4 * HID + 2, :]
    bias1 = bh1_ref[...] + jnp.dot(hf, A, preferred_element_type=jnp.float32,
                                   precision=_HIGH)
    z = (jnp.dot(ht, B, preferred_element_type=jnp.float32, precision=_HIGH)
         + jnp.dot(dabs, C, preferred_element_type=jnp.float32,
                   precision=_HIGH)
         + jnp.dot(prod, D, preferred_element_type=jnp.float32,
                   precision=_HIGH)
         + tn * wt + ir * wr + bias1)
    z = jnp.where(z > 0, z, jnp.exp(z) - 1.0)
    z = jnp.dot(z, wh2_ref[...], preferred_element_type=jnp.float32,
                precision=_HIGH) + bh2_ref[...]
    z = jnp.where(z > 0, z, jnp.exp(z) - 1.0)
    l = jnp.dot(z, wh3_ref[...], preferred_element_type=jnp.float32,
                precision=_HIGH) + bh3_ref[0, 0]
    log_ref[...] = l
    m = jnp.max(l)
    e = jnp.exp(l - m)
    prob_ref[...] = e / jnp.sum(e)


def _tc4(emb_pad, ht, tv, ir, wh1, bh1, wh2, bh2, wh3, bh3):
    return pl.pallas_call(
        _tc4_body,
        out_shape=(jax.ShapeDtypeStruct((N_ACT, 4 * HID + 2), jnp.float32),
                   jax.ShapeDtypeStruct((N_ACT, 1), jnp.float32),
                   jax.ShapeDtypeStruct((N_ACT, 1), jnp.float32)),
        interpret=_INTERPRET,
    )(emb_pad, ht, tv.reshape(N_ACT, 1), ir.reshape(N_ACT, 1),
      wh1, bh1.reshape(1, HID), wh2, bh2.reshape(1, HID), wh3,
      bh3.reshape(1, 1))


def _edge_arrays(edge_index):
    pad1 = jnp.full((E_SLOTS - 2 * N_EDGES,), PAD_IDX, jnp.int32)
    pad2 = jnp.full((NW, K_ALLOC - K_REAL, CHUNK), PAD_IDX, jnp.int32)
    s = jnp.concatenate([edge_index[0], edge_index[1], pad1])
    d = jnp.concatenate([edge_index[1], edge_index[0], pad1])
    s_arr = jnp.concatenate([s.reshape(NW, K_REAL, CHUNK), pad2], axis=1)
    d_arr = jnp.concatenate([d.reshape(NW, K_REAL, CHUNK), pad2], axis=1)
    return s_arr, d_arr


def kernel(x, edge_index, branch_child, time_value, is_root, W1, b1, W2, b2,
           Wh1, bh1, Wh2, bh2, Wh3, bh3):
    s_arr, d_arr = _edge_arrays(edge_index)
    z64 = jnp.zeros((N_PAD, HID), jnp.float32)
    z16 = jnp.zeros((N_PAD, 16), jnp.float32)
    ones16 = jnp.ones((CHUNK, 16), jnp.float32)

    dpart = _sc_deg(d_arr, ones16, z16)
    hws1, dinv = _tc1(x, W1, dpart)
    agg1 = _sc_spmm(s_arr, d_arr, hws1, z64)
    hws2 = _tc2(agg1, hws1, dinv, b1, W2)
    agg2 = _sc_spmm(s_arr, d_arr, hws2, z64)
    emb_pad = _tc3(agg2, hws2, dinv, b2)
    ht = _sc_gather(branch_child, emb_pad)
    ef, logits, probs = _tc4(emb_pad, ht, time_value, is_root,
                             Wh1, bh1, Wh2, bh2, Wh3, bh3)

    emb = emb_pad[:N_NODES + 1]
    leaf_feature = jnp.zeros((126,), jnp.float32).at[FOCAL].set(1.0)
    return (logits[:, 0], probs[:, 0], ef, emb, leaf_feature)


# speedup vs baseline: 46.5589x; 1.9619x over previous
import functools

import jax
import jax.numpy as jnp
from jax import lax
from jax.experimental import pallas as pl
from jax.experimental.pallas import tpu as pltpu
from jax.experimental.pallas import tpu_sc as plsc

N_NODES = 10000
N_PAD = 10112
PAD_IDX = N_NODES + 1
N_EDGES = 320000
HID = 64
HHALF = HID // 2
IN_DIM = 128
N_ACT = 4096
FOCAL = 5

NC = 2
NT = 16
ROWS_PER_TILE = N_PAD // NT
CHUNK = 128
DEGW = 16
K_REAL = 314
K_HALF = K_REAL // 2
K_ALLOC = 316
E_SLOTS = NT * K_REAL * CHUNK

_HIGH = jax.lax.Precision.HIGHEST
_INTERPRET = False

_SC_PARAMS = pltpu.CompilerParams(use_tc_tiling_on_sc=False)


def _sc_mesh():
    return plsc.VectorSubcoreMesh(core_axis_name="c", subcore_axis_name="s")


def _rowmask(nrows):
    r = lax.broadcasted_iota(jnp.int32, (nrows, 1), 0)
    return (r <= N_NODES).astype(jnp.float32)


def _sc_deg_body(d_hbm, ones_hbm, z_hbm, out_hbm, d_v, dbuf, ones_v, sem,
                 acc):
    cid = lax.axis_index("c")
    sid = lax.axis_index("s")
    row0 = sid * ROWS_PER_TILE
    pltpu.sync_copy(d_hbm.at[sid], d_v)
    pltpu.sync_copy(ones_hbm, ones_v)
    pltpu.sync_copy(z_hbm.at[pl.ds(row0, ROWS_PER_TILE)],
                    acc.at[pl.ds(row0, ROWS_PER_TILE)])
    plsc.subcore_barrier()

    def step(j, carry):
        jj = j + cid * K_HALF
        for i in range(CHUNK // 16):
            dbuf[pl.ds(16 * i, 16)] = d_v[jj, pl.ds(16 * i, 16)]
        pltpu.sync_copy(ones_v, acc.at[dbuf], add=True)
        return carry

    lax.fori_loop(0, K_HALF, step, 0)
    plsc.subcore_barrier()
    pltpu.sync_copy(acc.at[pl.ds(row0, ROWS_PER_TILE)],
                    out_hbm.at[cid, pl.ds(row0, ROWS_PER_TILE)])


def _sc_deg(d_arr, ones4, z4):
    return pl.kernel(
        _sc_deg_body,
        out_type=jax.ShapeDtypeStruct((NC, N_PAD, DEGW), jnp.float32),
        mesh=_sc_mesh(),
        compiler_params=_SC_PARAMS,
        scratch_types=[
            pltpu.VMEM((K_ALLOC, CHUNK), jnp.int32),
            pltpu.VMEM((CHUNK,), jnp.int32),
            pltpu.VMEM((CHUNK, DEGW), jnp.float32),
            pltpu.SemaphoreType.DMA,
            pltpu.VMEM_SHARED((N_PAD, DEGW), jnp.float32),
        ],
    )(d_arr, ones4, z4)


def _sc_spmm_body(s_hbm, d_hbm, tbl_hbm, z_hbm, out_hbm, s_v, d_v, dbuf,
                  buf_a, buf_b, sem_a, sem_b, acc, tbl_s):
    cid = lax.axis_index("c")
    sid = lax.axis_index("s")
    row0 = sid * ROWS_PER_TILE
    pltpu.sync_copy(s_hbm.at[sid], s_v)
    pltpu.sync_copy(d_hbm.at[sid], d_v)
    pltpu.sync_copy(z_hbm.at[pl.ds(row0, ROWS_PER_TILE)],
                    acc.at[pl.ds(row0, ROWS_PER_TILE)])
    pltpu.sync_copy(tbl_hbm.at[cid, pl.ds(row0, ROWS_PER_TILE)],
                    tbl_s.at[pl.ds(row0, ROWS_PER_TILE)])
    plsc.subcore_barrier()

    pltpu.async_copy(tbl_s.at[s_v.at[0]], buf_a, sem_a)

    def pair(g, carry):
        j0 = 2 * g
        j1 = j0 + 1
        pltpu.async_copy(tbl_s.at[s_v.at[j1]], buf_b, sem_b)
        pltpu.make_async_copy(tbl_s.at[s_v.at[j0]], buf_a, sem_a).wait()
        for i in range(CHUNK // 16):
            dbuf[pl.ds(16 * i, 16)] = d_v[j0, pl.ds(16 * i, 16)]
        pltpu.sync_copy(buf_a, acc.at[dbuf], add=True)
        pltpu.async_copy(tbl_s.at[s_v.at[j0 + 2]], buf_a, sem_a)
        pltpu.make_async_copy(tbl_s.at[s_v.at[j1]], buf_b, sem_b).wait()
        for i in range(CHUNK // 16):
            dbuf[pl.ds(16 * i, 16)] = d_v[j1, pl.ds(16 * i, 16)]
        pltpu.sync_copy(buf_b, acc.at[dbuf], add=True)
        return carry

    lax.fori_loop(0, K_REAL // 2, pair, 0)
    pltpu.make_async_copy(tbl_s.at[s_v.at[K_REAL]], buf_a, sem_a).wait()
    plsc.subcore_barrier()
    pltpu.sync_copy(acc.at[pl.ds(row0, ROWS_PER_TILE)],
                    out_hbm.at[cid, pl.ds(row0, ROWS_PER_TILE)])


def _sc_spmm(s_arr, d_arr, tbl, zh):
    return pl.kernel(
        _sc_spmm_body,
        out_type=jax.ShapeDtypeStruct((NC, N_PAD, HHALF), jnp.float32),
        mesh=_sc_mesh(),
        compiler_params=_SC_PARAMS,
        scratch_types=[
            pltpu.VMEM((K_ALLOC, CHUNK), jnp.int32),
            pltpu.VMEM((K_ALLOC, CHUNK), jnp.int32),
            pltpu.VMEM((CHUNK,), jnp.int32),
            pltpu.VMEM((CHUNK, HHALF), jnp.float32),
            pltpu.VMEM((CHUNK, HHALF), jnp.float32),
            pltpu.SemaphoreType.DMA,
            pltpu.SemaphoreType.DMA,
            pltpu.VMEM_SHARED((N_PAD, HHALF), jnp.float32),
            pltpu.VMEM_SHARED((N_PAD, HHALF), jnp.float32),
        ],
    )(s_arr, d_arr, tbl, zh)


def _sc_gather_body(bc_hbm, tbl_hbm, out_hbm, idx_v, buf, sem):
    cid = lax.axis_index("c")
    sid = lax.axis_index("s")
    wid = cid * NT + sid
    pltpu.sync_copy(bc_hbm.at[pl.ds(wid * CHUNK, CHUNK)], idx_v)
    pltpu.async_copy(tbl_hbm.at[idx_v], buf, sem).wait()
    pltpu.sync_copy(buf, out_hbm.at[pl.ds(wid * CHUNK, CHUNK)])


def _sc_gather(bc, tbl):
    return pl.kernel(
        _sc_gather_body,
        out_type=jax.ShapeDtypeStruct((N_ACT, HID), jnp.float32),
        mesh=_sc_mesh(),
        compiler_params=_SC_PARAMS,
        scratch_types=[
            pltpu.VMEM((CHUNK,), jnp.int32),
            pltpu.VMEM((CHUNK, HID), jnp.float32),
            pltpu.SemaphoreType.DMA,
        ],
    )(bc, tbl)


def _tc1_body(x_ref, w1_ref, dpart_ref, hws_ref, dinv_ref):
    deg = dpart_ref[0, :, 0:1] + dpart_ref[1, :, 0:1] + 1.0
    dinv = lax.rsqrt(jnp.clip(deg, 1.0, None))
    dinv_ref[...] = dinv
    hw = jnp.dot(x_ref[...], w1_ref[...], preferred_element_type=jnp.float32,
                 precision=_HIGH)
    sc = dinv[0:N_NODES] * hw
    hws_ref[0, 0:N_NODES, :] = sc[:, 0:HHALF]
    hws_ref[1, 0:N_NODES, :] = sc[:, HHALF:HID]
    foc = w1_ref[FOCAL:FOCAL + 1, :] + w1_ref[IN_DIM - 1:IN_DIM, :]
    focs = dinv[N_NODES:N_NODES + 1] * foc
    tail = lax.broadcasted_iota(jnp.int32, (N_PAD - N_NODES, 1), 0)
    focmask = (tail == 0).astype(jnp.float32)
    hws_ref[0, N_NODES:N_PAD, :] = focmask * focs[:, 0:HHALF]
    hws_ref[1, N_NODES:N_PAD, :] = focmask * focs[:, HHALF:HID]


def _tc1(x, w1, dpart):
    return pl.pallas_call(
        _tc1_body,
        out_shape=(jax.ShapeDtypeStruct((NC, N_PAD, HHALF), jnp.float32),
                   jax.ShapeDtypeStruct((N_PAD, 1), jnp.float32)),
        interpret=_INTERPRET,
    )(x, w1, dpart)


RBLK = 1264


def _blockmask():
    base = pl.program_id(0) * RBLK
    r = base + lax.broadcasted_iota(jnp.int32, (RBLK, 1), 0)
    return (r <= N_NODES).astype(jnp.float32)


def _tc2_body(agg_ref, hws1_ref, dinv_ref, b1_ref, w2_ref, hws2_ref):
    dinv = dinv_ref[...]
    mask = _blockmask()
    h_lo = jnp.maximum(dinv * (agg_ref[0] + hws1_ref[0])
                       + b1_ref[:, 0:HHALF], 0.0) * mask
    h_hi = jnp.maximum(dinv * (agg_ref[1] + hws1_ref[1])
                       + b1_ref[:, HHALF:HID], 0.0) * mask
    hw2 = (jnp.dot(h_lo, w2_ref[0:HHALF, :],
                   preferred_element_type=jnp.float32, precision=_HIGH)
           + jnp.dot(h_hi, w2_ref[HHALF:HID, :],
                     preferred_element_type=jnp.float32, precision=_HIGH))
    sc = dinv * hw2
    hws2_ref[0] = sc[:, 0:HHALF]
    hws2_ref[1] = sc[:, HHALF:HID]


def _tc2(agg, hws1, dinv, b1, w2):
    half = pl.BlockSpec((NC, RBLK, HHALF), lambda i: (0, i, 0))
    return pl.pallas_call(
        _tc2_body,
        grid=(N_PAD // RBLK,),
        in_specs=[half, half,
                  pl.BlockSpec((RBLK, 1), lambda i: (i, 0)),
                  pl.BlockSpec((1, HID), lambda i: (0, 0)),
                  pl.BlockSpec((HID, HID), lambda i: (0, 0))],
        out_specs=half,
        out_shape=jax.ShapeDtypeStruct((NC, N_PAD, HHALF), jnp.float32),
        interpret=_INTERPRET,
    )(agg, hws1, dinv, b1.reshape(1, HID), w2)


def _tc3_body(agg_ref, hws2_ref, dinv_ref, b2_ref, emb_ref):
    dinv = dinv_ref[...]
    mask = _blockmask()
    emb_ref[:, 0:HHALF] = (dinv * (agg_ref[0] + hws2_ref[0])
                           + b2_ref[:, 0:HHALF]) * mask
    emb_ref[:, HHALF:HID] = (dinv * (agg_ref[1] + hws2_ref[1])
                             + b2_ref[:, HHALF:HID]) * mask


def _tc3(agg, hws2, dinv, b2):
    half = pl.BlockSpec((NC, RBLK, HHALF), lambda i: (0, i, 0))
    return pl.pallas_call(
        _tc3_body,
        grid=(N_PAD // RBLK,),
        in_specs=[half, half,
                  pl.BlockSpec((RBLK, 1), lambda i: (i, 0)),
                  pl.BlockSpec((1, HID), lambda i: (0, 0))],
        out_specs=pl.BlockSpec((RBLK, HID), lambda i: (i, 0)),
        out_shape=jax.ShapeDtypeStruct((N_PAD, HID), jnp.float32),
        interpret=_INTERPRET,
    )(agg, hws2, dinv, b2.reshape(1, HID))


def _tc4_body(emb_ref, ht_ref, tn_ref, ir_ref, wh1_ref, bh1_ref, wh2_ref,
              bh2_ref, wh3_ref, bh3_ref, ef_ref, log_ref, prob_ref):
    hf = emb_ref[N_NODES:N_NODES + 1, :]
    ht = ht_ref[...]
    hfb = jnp.broadcast_to(hf, ht.shape)
    dabs = jnp.abs(hfb - ht)
    prod = hfb * ht
    tn = tn_ref[...] * (1.0 / (1.0 + 1e-08))
    ir = ir_ref[...]
    ef_ref[...] = jnp.concatenate([hfb, ht, dabs, prod, tn, ir], axis=1)

    A = wh1_ref[0:HID, :]
    B = wh1_ref[HID:2 * HID, :]
    C = wh1_ref[2 * HID:3 * HID, :]
    D = wh1_ref[3 * HID:4 * HID, :]
    wt = wh1_ref[4 * HID:4 * HID + 1, :]
    wr = wh1_ref[4 * HID + 1:4 * HID + 2, :]
    bias1 = bh1_ref[...] + jnp.dot(hf, A, preferred_element_type=jnp.float32,
                                   precision=_HIGH)
    z = (jnp.dot(ht, B, preferred_element_type=jnp.float32, precision=_HIGH)
         + jnp.dot(dabs, C, preferred_element_type=jnp.float32,
                   precision=_HIGH)
         + jnp.dot(prod, D, preferred_element_type=jnp.float32,
                   precision=_HIGH)
         + tn * wt + ir * wr + bias1)
    z = jnp.where(z > 0, z, jnp.exp(z) - 1.0)
    z = jnp.dot(z, wh2_ref[...], preferred_element_type=jnp.float32,
                precision=_HIGH) + bh2_ref[...]
    z = jnp.where(z > 0, z, jnp.exp(z) - 1.0)
    l = jnp.dot(z, wh3_ref[...], preferred_element_type=jnp.float32,
                precision=_HIGH) + bh3_ref[0, 0]
    log_ref[...] = l
    m = jnp.max(l)
    e = jnp.exp(l - m)
    prob_ref[...] = e / jnp.sum(e)


def _tc4(emb_pad, ht, tv, ir, wh1, bh1, wh2, bh2, wh3, bh3):
    return pl.pallas_call(
        _tc4_body,
        out_shape=(jax.ShapeDtypeStruct((N_ACT, 4 * HID + 2), jnp.float32),
                   jax.ShapeDtypeStruct((N_ACT, 1), jnp.float32),
                   jax.ShapeDtypeStruct((N_ACT, 1), jnp.float32)),
        interpret=_INTERPRET,
    )(emb_pad, ht, tv.reshape(N_ACT, 1), ir.reshape(N_ACT, 1),
      wh1, bh1.reshape(1, HID), wh2, bh2.reshape(1, HID), wh3,
      bh3.reshape(1, 1))


def _edge_arrays(edge_index):
    pad1 = jnp.full((E_SLOTS - 2 * N_EDGES,), PAD_IDX, jnp.int32)
    pad2 = jnp.full((NT, K_ALLOC - K_REAL, CHUNK), PAD_IDX, jnp.int32)
    s = jnp.concatenate([edge_index[0], edge_index[1], pad1])
    d = jnp.concatenate([edge_index[1], edge_index[0], pad1])
    s_arr = jnp.concatenate([s.reshape(NT, K_REAL, CHUNK), pad2], axis=1)
    d_arr = jnp.concatenate([d.reshape(NT, K_REAL, CHUNK), pad2], axis=1)
    return s_arr, d_arr


def kernel(x, edge_index, branch_child, time_value, is_root, W1, b1, W2, b2,
           Wh1, bh1, Wh2, bh2, Wh3, bh3):
    s_arr, d_arr = _edge_arrays(edge_index)
    zh = jnp.zeros((N_PAD, HHALF), jnp.float32)
    z4 = jnp.zeros((N_PAD, DEGW), jnp.float32)
    ones4 = jnp.ones((CHUNK, DEGW), jnp.float32)

    dpart = _sc_deg(d_arr, ones4, z4)
    hws1, dinv = _tc1(x, W1, dpart)
    agg1 = _sc_spmm(s_arr, d_arr, hws1, zh)
    hws2 = _tc2(agg1, hws1, dinv, b1, W2)
    agg2 = _sc_spmm(s_arr, d_arr, hws2, zh)
    emb_pad = _tc3(agg2, hws2, dinv, b2)
    ht = _sc_gather(branch_child, emb_pad)
    ef, logits, probs = _tc4(emb_pad, ht, time_value, is_root,
                             Wh1, bh1, Wh2, bh2, Wh3, bh3)

    emb = emb_pad[:N_NODES + 1]
    leaf_feature = jnp.zeros((126,), jnp.float32).at[FOCAL].set(1.0)
    return (logits[:, 0], probs[:, 0], ef, emb, leaf_feature)


# speedup vs baseline: 48.4531x; 1.0407x over previous
import functools

import jax
import jax.numpy as jnp
from jax import lax
from jax.experimental import pallas as pl
from jax.experimental.pallas import tpu as pltpu
from jax.experimental.pallas import tpu_sc as plsc

N_NODES = 10000
N_PAD = 10112
PAD_IDX = N_NODES + 1
N_EDGES = 320000
HID = 64
HHALF = HID // 2
IN_DIM = 128
N_ACT = 4096
FOCAL = 5

NC = 2
NT = 16
ROWS_PER_TILE = N_PAD // NT
CHUNK = 128
DEGW = 16
K_REAL = 314
K_HALF = K_REAL // 2
K_ALLOC = 316
E_SLOTS = NT * K_REAL * CHUNK

_INTERPRET = False

_SC_PARAMS = pltpu.CompilerParams(use_tc_tiling_on_sc=False)


def _sc_mesh():
    return plsc.VectorSubcoreMesh(core_axis_name="c", subcore_axis_name="s")


def _rowmask(nrows):
    r = lax.broadcasted_iota(jnp.int32, (nrows, 1), 0)
    return (r <= N_NODES).astype(jnp.float32)


def _sc_deg_body(d_hbm, ones_hbm, z_hbm, out_hbm, d_v, dbuf, ones_v, sem,
                 acc):
    cid = lax.axis_index("c")
    sid = lax.axis_index("s")
    row0 = sid * ROWS_PER_TILE
    pltpu.sync_copy(d_hbm.at[sid], d_v)
    pltpu.sync_copy(ones_hbm, ones_v)
    pltpu.sync_copy(z_hbm.at[pl.ds(row0, ROWS_PER_TILE)],
                    acc.at[pl.ds(row0, ROWS_PER_TILE)])
    plsc.subcore_barrier()

    def step(j, carry):
        jj = j + cid * K_HALF
        for i in range(CHUNK // 16):
            dbuf[pl.ds(16 * i, 16)] = d_v[jj, pl.ds(16 * i, 16)]
        pltpu.sync_copy(ones_v, acc.at[dbuf], add=True)
        return carry

    lax.fori_loop(0, K_HALF, step, 0)
    plsc.subcore_barrier()
    pltpu.sync_copy(acc.at[pl.ds(row0, ROWS_PER_TILE)],
                    out_hbm.at[cid, pl.ds(row0, ROWS_PER_TILE)])


def _sc_deg(d_arr, ones4, z4):
    return pl.kernel(
        _sc_deg_body,
        out_type=jax.ShapeDtypeStruct((NC, N_PAD, DEGW), jnp.float32),
        mesh=_sc_mesh(),
        compiler_params=_SC_PARAMS,
        scratch_types=[
            pltpu.VMEM((K_ALLOC, CHUNK), jnp.int32),
            pltpu.VMEM((CHUNK,), jnp.int32),
            pltpu.VMEM((CHUNK, DEGW), jnp.float32),
            pltpu.SemaphoreType.DMA,
            pltpu.VMEM_SHARED((N_PAD, DEGW), jnp.float32),
        ],
    )(d_arr, ones4, z4)


def _sc_spmm_body(s_hbm, d_hbm, tbl_hbm, z_hbm, out_hbm, s_v, d_v, dbuf,
                  buf_a, buf_b, sem_a, sem_b, acc, tbl_s):
    cid = lax.axis_index("c")
    sid = lax.axis_index("s")
    row0 = sid * ROWS_PER_TILE
    pltpu.sync_copy(s_hbm.at[sid], s_v)
    pltpu.sync_copy(d_hbm.at[sid], d_v)
    pltpu.sync_copy(z_hbm.at[pl.ds(row0, ROWS_PER_TILE)],
                    acc.at[pl.ds(row0, ROWS_PER_TILE)])
    pltpu.sync_copy(tbl_hbm.at[cid, pl.ds(row0, ROWS_PER_TILE)],
                    tbl_s.at[pl.ds(row0, ROWS_PER_TILE)])
    plsc.subcore_barrier()

    pltpu.async_copy(tbl_s.at[s_v.at[0]], buf_a, sem_a)

    def pair(g, carry):
        j0 = 2 * g
        j1 = j0 + 1
        pltpu.async_copy(tbl_s.at[s_v.at[j1]], buf_b, sem_b)
        pltpu.make_async_copy(tbl_s.at[s_v.at[j0]], buf_a, sem_a).wait()
        for i in range(CHUNK // 16):
            dbuf[pl.ds(16 * i, 16)] = d_v[j0, pl.ds(16 * i, 16)]
        pltpu.sync_copy(buf_a, acc.at[dbuf], add=True)
        pltpu.async_copy(tbl_s.at[s_v.at[j0 + 2]], buf_a, sem_a)
        pltpu.make_async_copy(tbl_s.at[s_v.at[j1]], buf_b, sem_b).wait()
        for i in range(CHUNK // 16):
            dbuf[pl.ds(16 * i, 16)] = d_v[j1, pl.ds(16 * i, 16)]
        pltpu.sync_copy(buf_b, acc.at[dbuf], add=True)
        return carry

    lax.fori_loop(0, K_REAL // 2, pair, 0)
    pltpu.make_async_copy(tbl_s.at[s_v.at[K_REAL]], buf_a, sem_a).wait()
    plsc.subcore_barrier()
    pltpu.sync_copy(acc.at[pl.ds(row0, ROWS_PER_TILE)],
                    out_hbm.at[cid, pl.ds(row0, ROWS_PER_TILE)])


def _sc_spmm(s_arr, d_arr, tbl, zh):
    return pl.kernel(
        _sc_spmm_body,
        out_type=jax.ShapeDtypeStruct((NC, N_PAD, HHALF), jnp.float32),
        mesh=_sc_mesh(),
        compiler_params=_SC_PARAMS,
        scratch_types=[
            pltpu.VMEM((K_ALLOC, CHUNK), jnp.int32),
            pltpu.VMEM((K_ALLOC, CHUNK), jnp.int32),
            pltpu.VMEM((CHUNK,), jnp.int32),
            pltpu.VMEM((CHUNK, HHALF), jnp.float32),
            pltpu.VMEM((CHUNK, HHALF), jnp.float32),
            pltpu.SemaphoreType.DMA,
            pltpu.SemaphoreType.DMA,
            pltpu.VMEM_SHARED((N_PAD, HHALF), jnp.float32),
            pltpu.VMEM_SHARED((N_PAD, HHALF), jnp.float32),
        ],
    )(s_arr, d_arr, tbl, zh)


def _sc_gather_body(bc_hbm, tbl_hbm, out_hbm, idx_v, buf, sem):
    cid = lax.axis_index("c")
    sid = lax.axis_index("s")
    wid = cid * NT + sid
    pltpu.sync_copy(bc_hbm.at[pl.ds(wid * CHUNK, CHUNK)], idx_v)
    pltpu.async_copy(tbl_hbm.at[idx_v], buf, sem).wait()
    pltpu.sync_copy(buf, out_hbm.at[pl.ds(wid * CHUNK, CHUNK)])


def _sc_gather(bc, tbl):
    return pl.kernel(
        _sc_gather_body,
        out_type=jax.ShapeDtypeStruct((N_ACT, HID), jnp.float32),
        mesh=_sc_mesh(),
        compiler_params=_SC_PARAMS,
        scratch_types=[
            pltpu.VMEM((CHUNK,), jnp.int32),
            pltpu.VMEM((CHUNK, HID), jnp.float32),
            pltpu.SemaphoreType.DMA,
        ],
    )(bc, tbl)


def _tc1_body(x_ref, w1_ref, dpart_ref, hws_ref, dinv_ref):
    deg = dpart_ref[0, :, 0:1] + dpart_ref[1, :, 0:1] + 1.0
    dinv = lax.rsqrt(jnp.clip(deg, 1.0, None))
    dinv_ref[...] = dinv
    hw = jnp.dot(x_ref[...], w1_ref[...], preferred_element_type=jnp.float32)
    sc = dinv[0:N_NODES] * hw
    hws_ref[0, 0:N_NODES, :] = sc[:, 0:HHALF]
    hws_ref[1, 0:N_NODES, :] = sc[:, HHALF:HID]
    w5 = w1_ref[FOCAL:FOCAL + 1, :].astype(jnp.bfloat16).astype(jnp.float32)
    wl = (w1_ref[IN_DIM - 1:IN_DIM, :].astype(jnp.bfloat16)
          .astype(jnp.float32))
    foc = w5 + wl
    focs = dinv[N_NODES:N_NODES + 1] * foc
    tail = lax.broadcasted_iota(jnp.int32, (N_PAD - N_NODES, 1), 0)
    focmask = (tail == 0).astype(jnp.float32)
    hws_ref[0, N_NODES:N_PAD, :] = focmask * focs[:, 0:HHALF]
    hws_ref[1, N_NODES:N_PAD, :] = focmask * focs[:, HHALF:HID]


def _tc1(x, w1, dpart):
    return pl.pallas_call(
        _tc1_body,
        out_shape=(jax.ShapeDtypeStruct((NC, N_PAD, HHALF), jnp.float32),
                   jax.ShapeDtypeStruct((N_PAD, 1), jnp.float32)),
        interpret=_INTERPRET,
    )(x, w1, dpart)


RBLK = 1264


def _blockmask():
    base = pl.program_id(0) * RBLK
    r = base + lax.broadcasted_iota(jnp.int32, (RBLK, 1), 0)
    return (r <= N_NODES).astype(jnp.float32)


def _tc2_body(agg_ref, hws1_ref, dinv_ref, b1_ref, w2_ref, hws2_ref):
    dinv = dinv_ref[...]
    mask = _blockmask()
    h_lo = jnp.maximum(dinv * (agg_ref[0] + hws1_ref[0])
                       + b1_ref[:, 0:HHALF], 0.0) * mask
    h_hi = jnp.maximum(dinv * (agg_ref[1] + hws1_ref[1])
                       + b1_ref[:, HHALF:HID], 0.0) * mask
    hw2 = (jnp.dot(h_lo, w2_ref[0:HHALF, :],
                   preferred_element_type=jnp.float32)
           + jnp.dot(h_hi, w2_ref[HHALF:HID, :],
                     preferred_element_type=jnp.float32))
    sc = dinv * hw2
    hws2_ref[0] = sc[:, 0:HHALF]
    hws2_ref[1] = sc[:, HHALF:HID]


def _tc2(agg, hws1, dinv, b1, w2):
    half = pl.BlockSpec((NC, RBLK, HHALF), lambda i: (0, i, 0))
    return pl.pallas_call(
        _tc2_body,
        grid=(N_PAD // RBLK,),
        in_specs=[half, half,
                  pl.BlockSpec((RBLK, 1), lambda i: (i, 0)),
                  pl.BlockSpec((1, HID), lambda i: (0, 0)),
                  pl.BlockSpec((HID, HID), lambda i: (0, 0))],
        out_specs=half,
        out_shape=jax.ShapeDtypeStruct((NC, N_PAD, HHALF), jnp.float32),
        interpret=_INTERPRET,
    )(agg, hws1, dinv, b1.reshape(1, HID), w2)


def _tc3_body(agg_ref, hws2_ref, dinv_ref, b2_ref, emb_ref):
    dinv = dinv_ref[...]
    mask = _blockmask()
    emb_ref[:, 0:HHALF] = (dinv * (agg_ref[0] + hws2_ref[0])
                           + b2_ref[:, 0:HHALF]) * mask
    emb_ref[:, HHALF:HID] = (dinv * (agg_ref[1] + hws2_ref[1])
                             + b2_ref[:, HHALF:HID]) * mask


def _tc3(agg, hws2, dinv, b2):
    half = pl.BlockSpec((NC, RBLK, HHALF), lambda i: (0, i, 0))
    return pl.pallas_call(
        _tc3_body,
        grid=(N_PAD // RBLK,),
        in_specs=[half, half,
                  pl.BlockSpec((RBLK, 1), lambda i: (i, 0)),
                  pl.BlockSpec((1, HID), lambda i: (0, 0))],
        out_specs=pl.BlockSpec((RBLK, HID), lambda i: (i, 0)),
        out_shape=jax.ShapeDtypeStruct((N_PAD, HID), jnp.float32),
        interpret=_INTERPRET,
    )(agg, hws2, dinv, b2.reshape(1, HID))


def _tc4_body(emb_ref, ht_ref, tn_ref, ir_ref, wh1_ref, bh1_ref, wh2_ref,
              bh2_ref, wh3_ref, bh3_ref, ef_ref, log_ref, prob_ref):
    hf = emb_ref[N_NODES:N_NODES + 1, :]
    ht = ht_ref[...]
    hfb = jnp.broadcast_to(hf, ht.shape)
    dabs = jnp.abs(hfb - ht)
    prod = hfb * ht
    tn = tn_ref[...] * (1.0 / (1.0 + 1e-08))
    ir = ir_ref[...]
    ef_ref[...] = jnp.concatenate([hfb, ht, dabs, prod, tn, ir], axis=1)

    A = wh1_ref[0:HID, :]
    B = wh1_ref[HID:2 * HID, :]
    C = wh1_ref[2 * HID:3 * HID, :]
    D = wh1_ref[3 * HID:4 * HID, :]
    wt = wh1_ref[4 * HID:4 * HID + 1, :]
    wr = wh1_ref[4 * HID + 1:4 * HID + 2, :]
    bias1 = bh1_ref[...] + jnp.dot(hf, A, preferred_element_type=jnp.float32)
    z = (jnp.dot(ht, B, preferred_element_type=jnp.float32)
         + jnp.dot(dabs, C, preferred_element_type=jnp.float32)
         + jnp.dot(prod, D, preferred_element_type=jnp.float32)
         + tn.astype(jnp.bfloat16).astype(jnp.float32)
         * wt.astype(jnp.bfloat16).astype(jnp.float32)
         + ir.astype(jnp.bfloat16).astype(jnp.float32)
         * wr.astype(jnp.bfloat16).astype(jnp.float32) + bias1)
    z = jnp.where(z > 0, z, jnp.exp(z) - 1.0)
    z = jnp.dot(z, wh2_ref[...], preferred_element_type=jnp.float32) + bh2_ref[...]
    z = jnp.where(z > 0, z, jnp.exp(z) - 1.0)
    l = jnp.dot(z, wh3_ref[...], preferred_element_type=jnp.float32) + bh3_ref[0, 0]
    log_ref[...] = l
    m = jnp.max(l)
    e = jnp.exp(l - m)
    prob_ref[...] = e / jnp.sum(e)


def _tc4(emb_pad, ht, tv, ir, wh1, bh1, wh2, bh2, wh3, bh3):
    return pl.pallas_call(
        _tc4_body,
        out_shape=(jax.ShapeDtypeStruct((N_ACT, 4 * HID + 2), jnp.float32),
                   jax.ShapeDtypeStruct((N_ACT, 1), jnp.float32),
                   jax.ShapeDtypeStruct((N_ACT, 1), jnp.float32)),
        interpret=_INTERPRET,
    )(emb_pad, ht, tv.reshape(N_ACT, 1), ir.reshape(N_ACT, 1),
      wh1, bh1.reshape(1, HID), wh2, bh2.reshape(1, HID), wh3,
      bh3.reshape(1, 1))


def _edge_arrays(edge_index):
    pad1 = jnp.full((E_SLOTS - 2 * N_EDGES,), PAD_IDX, jnp.int32)
    pad2 = jnp.full((NT, K_ALLOC - K_REAL, CHUNK), PAD_IDX, jnp.int32)
    s = jnp.concatenate([edge_index[0], edge_index[1], pad1])
    d = jnp.concatenate([edge_index[1], edge_index[0], pad1])
    s_arr = jnp.concatenate([s.reshape(NT, K_REAL, CHUNK), pad2], axis=1)
    d_arr = jnp.concatenate([d.reshape(NT, K_REAL, CHUNK), pad2], axis=1)
    return s_arr, d_arr


def kernel(x, edge_index, branch_child, time_value, is_root, W1, b1, W2, b2,
           Wh1, bh1, Wh2, bh2, Wh3, bh3):
    s_arr, d_arr = _edge_arrays(edge_index)
    zh = jnp.zeros((N_PAD, HHALF), jnp.float32)
    z4 = jnp.zeros((N_PAD, DEGW), jnp.float32)
    ones4 = jnp.ones((CHUNK, DEGW), jnp.float32)

    dpart = _sc_deg(d_arr, ones4, z4)
    hws1, dinv = _tc1(x, W1, dpart)
    agg1 = _sc_spmm(s_arr, d_arr, hws1, zh)
    hws2 = _tc2(agg1, hws1, dinv, b1, W2)
    agg2 = _sc_spmm(s_arr, d_arr, hws2, zh)
    emb_pad = _tc3(agg2, hws2, dinv, b2)
    ht = _sc_gather(branch_child, emb_pad)
    ef, logits, probs = _tc4(emb_pad, ht, time_value, is_root,
                             Wh1, bh1, Wh2, bh2, Wh3, bh3)

    emb = emb_pad[:N_NODES + 1]
    leaf_feature = jnp.zeros((126,), jnp.float32).at[FOCAL].set(1.0)
    return (logits[:, 0], probs[:, 0], ef, emb, leaf_feature)
